# baseline (speedup 1.0000x reference)
import functools

import jax
import jax.numpy as jnp
from jax import lax
from jax.experimental import pallas as pl
from jax.experimental.pallas import tpu as pltpu
from jax.experimental.pallas import tpu_sc as plsc

_C = 100000
_N = 1024
_R = 8
_SMOOTH = 0.1
_CONF = 1.0 - _SMOOTH
_EPS = _SMOOTH / (_C - 1)

_NC = 2
_NS = 16
_NW = _NC * _NS
_RW = _N // _NW
_L = 16


def _tc_body(pred_ref, out_ref):
    i = pl.program_id(0)
    x = pred_ref[...]
    m = jnp.max(x, axis=1, keepdims=True)
    se = jnp.sum(jnp.exp(x - m), axis=1, keepdims=True)
    lse = m + jnp.log(se)
    sx = jnp.sum(x, axis=1, keepdims=True)
    row_loss = _EPS * (_C * lse - sx) + (_CONF - _EPS) * lse
    part = jnp.sum(row_loss) * (1.0 / _N)

    @pl.when(i == 0)
    def _init():
        out_ref[0, 0] = 0.0

    out_ref[0, 0] += part


@functools.partial(
    pl.kernel,
    mesh=plsc.VectorSubcoreMesh(core_axis_name="c", subcore_axis_name="s"),
    out_type=jax.ShapeDtypeStruct((_NC, _L), jnp.float32),
    scratch_types=[
        pltpu.VMEM((_RW,), jnp.int32),
        pltpu.VMEM((_RW,), jnp.int32),
        pltpu.VMEM((_RW,), jnp.float32),
        pltpu.VMEM((_L,), jnp.float32),
        pltpu.VMEM((_NS, _L), jnp.float32),
        pltpu.VMEM_SHARED((_NS, _L), jnp.float32),
        pltpu.SemaphoreType.DMA,
    ],
)
def _sc_gather(pred1, tgt, out, tgt_v, flat_v, vals_v, acc_v,
               stage_v, shared, sem):
    c = lax.axis_index("c")
    s = lax.axis_index("s")
    wid = s * _NC + c
    base = wid * _RW
    pltpu.sync_copy(tgt.at[pl.ds(base, _RW)], tgt_v)
    for g in range(_RW // _L):
        t16 = tgt_v[pl.ds(g * _L, _L)]
        rows = base + g * _L + lax.iota(jnp.int32, _L)
        flat_v[pl.ds(g * _L, _L)] = rows * _C + t16
    pltpu.async_copy(pred1.at[flat_v], vals_v, sem).wait()
    acc = jnp.zeros((_L,), jnp.float32)
    for g in range(_RW // _L):
        acc = acc + vals_v[pl.ds(g * _L, _L)]
    acc_v[...] = acc
    pltpu.sync_copy(acc_v, shared.at[s])
    plsc.subcore_barrier()

    @pl.when(s == 0)
    def _reduce():
        pltpu.sync_copy(shared, stage_v)
        tot = jnp.zeros((_L,), jnp.float32)
        for j in range(_NS):
            tot = tot + stage_v[j]
        acc_v[...] = tot
        pltpu.sync_copy(acc_v, out.at[c])


def kernel(pred, target):
    tgt = target.astype(jnp.int32)
    dense = pl.pallas_call(
        _tc_body,
        grid=(_N // _R,),
        in_specs=[pl.BlockSpec((_R, _C), lambda i: (i, 0))],
        out_specs=pl.BlockSpec(memory_space=pltpu.SMEM,
                               block_shape=(1, 1),
                               index_map=lambda i: (0, 0)),
        out_shape=jax.ShapeDtypeStruct((1, 1), jnp.float32),
        compiler_params=pltpu.CompilerParams(
            dimension_semantics=("arbitrary",),
        ),
    )(pred)
    p1 = pred.reshape(-1)
    partials = _sc_gather(p1, tgt)
    bsum = jnp.sum(partials)
    return dense[0, 0] - (_CONF - _EPS) * (bsum * (1.0 / _N))


# speedup vs baseline: 6.7849x; 6.7849x over previous
import functools

import jax
import jax.numpy as jnp
from jax import lax
from jax.experimental import pallas as pl
from jax.experimental.pallas import tpu as pltpu
from jax.experimental.pallas import tpu_sc as plsc

_C = 100000
_N = 1024
_BK = 2000
_SMOOTH = 0.1
_CONF = 1.0 - _SMOOTH
_EPS = _SMOOTH / (_C - 1)

_NC = 2
_NS = 16
_NW = _NC * _NS
_RW = _N // _NW
_L = 16


def _tc_body(xt_ref, out_ref, m_ref, s_ref, sx_ref):
    k = pl.program_id(0)
    b = xt_ref[...]

    @pl.when(k == 0)
    def _init():
        m_ref[...] = jnp.full((1, _N), -jnp.inf, jnp.float32)
        s_ref[...] = jnp.zeros((1, _N), jnp.float32)
        sx_ref[...] = jnp.zeros((1, _N), jnp.float32)

    bm = jnp.max(b, axis=0, keepdims=True)
    m_old = m_ref[...]
    m_new = jnp.maximum(m_old, bm)
    es = jnp.sum(jnp.exp(b - m_new), axis=0, keepdims=True)
    s_ref[...] = s_ref[...] * jnp.exp(m_old - m_new) + es
    m_ref[...] = m_new
    sx_ref[...] += jnp.sum(b, axis=0, keepdims=True)

    @pl.when(k == (_C // _BK) - 1)
    def _fin():
        lse = m_ref[...] + jnp.log(s_ref[...])
        row_loss = _EPS * (_C * lse - sx_ref[...]) + (_CONF - _EPS) * lse
        out_ref[0, 0] = jnp.sum(row_loss) * (1.0 / _N)


@functools.partial(
    pl.kernel,
    mesh=plsc.VectorSubcoreMesh(core_axis_name="c", subcore_axis_name="s"),
    out_type=jax.ShapeDtypeStruct((_NC, _L), jnp.float32),
    scratch_types=[
        pltpu.VMEM((_RW,), jnp.int32),
        pltpu.VMEM((_RW,), jnp.int32),
        pltpu.VMEM((_RW, 128), jnp.float32),
        pltpu.VMEM((_L,), jnp.float32),
        pltpu.VMEM((_NS, _L), jnp.float32),
        pltpu.VMEM_SHARED((_NS, _L), jnp.float32),
        pltpu.SemaphoreType.DMA,
    ],
)
def _sc_gather(phys, tgt, out, tgt_v, idx_v, spans_v, acc_v,
               stage_v, shared, sem):
    c = lax.axis_index("c")
    s = lax.axis_index("s")
    wid = s * _NC + c
    base = wid * _RW
    pltpu.sync_copy(tgt.at[pl.ds(base, _RW)], tgt_v)
    for g in range(_RW // _L):
        t16 = tgt_v[pl.ds(g * _L, _L)]
        rows = base + g * _L + lax.iota(jnp.int32, _L)
        span = (lax.shift_right_logical(t16, 3) * 64
                + lax.shift_right_logical(rows, 7) * 8
                + lax.bitwise_and(t16, 7))
        idx_v[pl.ds(g * _L, _L)] = span
    pltpu.async_copy(phys.at[idx_v], spans_v, sem).wait()
    lane0 = lax.bitwise_and(wid, 3) * 32
    lane16 = lax.iota(jnp.int32, _L)
    acc = jnp.zeros((_L,), jnp.float32)
    for j in range(_RW):
        chunk = spans_v[j, pl.ds(lane0 + (j // _L) * _L, _L)]
        acc = acc + jnp.where(lane16 == (j % _L), chunk, 0.0)
    acc_v[...] = acc
    pltpu.sync_copy(acc_v, shared.at[s])
    plsc.subcore_barrier()

    @pl.when(s == 0)
    def _reduce():
        pltpu.sync_copy(shared, stage_v)
        tot = jnp.zeros((_L,), jnp.float32)
        for j in range(_NS):
            tot = tot + stage_v[j]
        acc_v[...] = tot
        pltpu.sync_copy(acc_v, out.at[c])


def kernel(pred, target):
    tgt = target.astype(jnp.int32)
    xt = pred.T
    dense = pl.pallas_call(
        _tc_body,
        grid=(_C // _BK,),
        in_specs=[pl.BlockSpec((_BK, _N), lambda k: (k, 0))],
        out_specs=pl.BlockSpec(memory_space=pltpu.SMEM,
                               block_shape=(1, 1),
                               index_map=lambda k: (0, 0)),
        out_shape=jax.ShapeDtypeStruct((1, 1), jnp.float32),
        scratch_shapes=[
            pltpu.VMEM((1, _N), jnp.float32),
            pltpu.VMEM((1, _N), jnp.float32),
            pltpu.VMEM((1, _N), jnp.float32),
        ],
        compiler_params=pltpu.CompilerParams(
            dimension_semantics=("arbitrary",),
        ),
    )(xt)
    phys = jnp.transpose(pred.reshape(8, 128, _C // 8, 8),
                         (2, 0, 3, 1)).reshape(_C * 8, 128)
    partials = _sc_gather(phys, tgt)
    bsum = jnp.sum(partials)
    return dense[0, 0] - (_CONF - _EPS) * (bsum * (1.0 / _N))


# speedup vs baseline: 7.3116x; 1.0776x over previous
import functools

import jax
import jax.numpy as jnp
from jax import lax
from jax.experimental import pallas as pl
from jax.experimental.pallas import tpu as pltpu
from jax.experimental.pallas import tpu_sc as plsc

_C = 100000
_N = 1024
_BK = 4000
_SMOOTH = 0.1
_CONF = 1.0 - _SMOOTH
_EPS = _SMOOTH / (_C - 1)

_NC = 2
_NS = 16
_NW = _NC * _NS
_RW = _N // _NW
_L = 16


def _tc_body(xt_ref, out_ref, m_ref, s_ref, sx_ref):
    k = pl.program_id(0)
    b = xt_ref[...]

    @pl.when(k == 0)
    def _init():
        m_ref[...] = jnp.full((1, _N), -jnp.inf, jnp.float32)
        s_ref[...] = jnp.zeros((1, _N), jnp.float32)
        sx_ref[...] = jnp.zeros((1, _N), jnp.float32)

    bm = jnp.max(b, axis=0, keepdims=True)
    m_old = m_ref[...]
    m_new = jnp.maximum(m_old, bm)
    es = jnp.sum(jnp.exp(b - m_new), axis=0, keepdims=True)
    s_ref[...] = s_ref[...] * jnp.exp(m_old - m_new) + es
    m_ref[...] = m_new
    sx_ref[...] += jnp.sum(b, axis=0, keepdims=True)

    @pl.when(k == (_C // _BK) - 1)
    def _fin():
        lse = m_ref[...] + jnp.log(s_ref[...])
        row_loss = _EPS * (_C * lse - sx_ref[...]) + (_CONF - _EPS) * lse
        out_ref[0, 0] = jnp.sum(row_loss) * (1.0 / _N)


@functools.partial(
    pl.kernel,
    mesh=plsc.VectorSubcoreMesh(core_axis_name="c", subcore_axis_name="s"),
    out_type=jax.ShapeDtypeStruct((_NC, _L), jnp.float32),
    scratch_types=[
        pltpu.VMEM((_RW,), jnp.int32),
        pltpu.VMEM((_RW,), jnp.int32),
        pltpu.VMEM((_RW, 128), jnp.float32),
        pltpu.VMEM((_L,), jnp.float32),
        pltpu.VMEM((_NS, _L), jnp.float32),
        pltpu.VMEM_SHARED((_NS, _L), jnp.float32),
        pltpu.SemaphoreType.DMA,
    ],
)
def _sc_gather(phys, tgt, out, tgt_v, idx_v, spans_v, acc_v,
               stage_v, shared, sem):
    c = lax.axis_index("c")
    s = lax.axis_index("s")
    wid = s * _NC + c
    base = wid * _RW
    pltpu.sync_copy(tgt.at[pl.ds(base, _RW)], tgt_v)
    for g in range(_RW // _L):
        t16 = tgt_v[pl.ds(g * _L, _L)]
        rows = base + g * _L + lax.iota(jnp.int32, _L)
        span = (lax.shift_right_logical(t16, 3) * 64
                + lax.shift_right_logical(rows, 7) * 8
                + lax.bitwise_and(t16, 7))
        idx_v[pl.ds(g * _L, _L)] = span
    pltpu.async_copy(phys.at[idx_v], spans_v, sem).wait()
    lane0 = lax.bitwise_and(wid, 3) * 32
    lane16 = lax.iota(jnp.int32, _L)
    acc = jnp.zeros((_L,), jnp.float32)
    for j in range(_RW):
        chunk = spans_v[j, pl.ds(lane0 + (j // _L) * _L, _L)]
        acc = acc + jnp.where(lane16 == (j % _L), chunk, 0.0)
    acc_v[...] = acc
    pltpu.sync_copy(acc_v, shared.at[s])
    plsc.subcore_barrier()

    @pl.when(s == 0)
    def _reduce():
        pltpu.sync_copy(shared, stage_v)
        tot = jnp.zeros((_L,), jnp.float32)
        for j in range(_NS):
            tot = tot + stage_v[j]
        acc_v[...] = tot
        pltpu.sync_copy(acc_v, out.at[c])


def kernel(pred, target):
    tgt = target.astype(jnp.int32)
    xt = pred.T
    dense = pl.pallas_call(
        _tc_body,
        grid=(_C // _BK,),
        in_specs=[pl.BlockSpec((_BK, _N), lambda k: (k, 0))],
        out_specs=pl.BlockSpec(memory_space=pltpu.SMEM,
                               block_shape=(1, 1),
                               index_map=lambda k: (0, 0)),
        out_shape=jax.ShapeDtypeStruct((1, 1), jnp.float32),
        scratch_shapes=[
            pltpu.VMEM((1, _N), jnp.float32),
            pltpu.VMEM((1, _N), jnp.float32),
            pltpu.VMEM((1, _N), jnp.float32),
        ],
        compiler_params=pltpu.CompilerParams(
            dimension_semantics=("arbitrary",),
        ),
    )(xt)
    phys = jnp.transpose(pred.reshape(8, 128, _C // 8, 8),
                         (2, 0, 3, 1)).reshape(_C * 8, 128)
    partials = _sc_gather(phys, tgt)
    bsum = jnp.sum(partials)
    return dense[0, 0] - (_CONF - _EPS) * (bsum * (1.0 / _N))


# speedup vs baseline: 7.8799x; 1.0777x over previous
import functools

import jax
import jax.numpy as jnp
from jax import lax
from jax.experimental import pallas as pl
from jax.experimental.pallas import tpu as pltpu
from jax.experimental.pallas import tpu_sc as plsc

_C = 100000
_N = 1024
_BK = 5000
_SUB = 1000
_SMOOTH = 0.1
_CONF = 1.0 - _SMOOTH
_EPS = _SMOOTH / (_C - 1)

_NC = 2
_NS = 16
_NW = _NC * _NS
_RW = _N // _NW
_L = 16


def _tc_body(xt_ref, out_ref, m_ref, s_ref, sx_ref):
    k = pl.program_id(0)

    @pl.when(k == 0)
    def _init():
        m_ref[...] = jnp.full((1, _N), -jnp.inf, jnp.float32)
        s_ref[...] = jnp.zeros((1, _N), jnp.float32)
        sx_ref[...] = jnp.zeros((1, _N), jnp.float32)

    for u in range(_BK // _SUB):
        b = xt_ref[pl.ds(u * _SUB, _SUB), :]
        bm = jnp.max(b, axis=0, keepdims=True)
        m_old = m_ref[...]
        m_new = jnp.maximum(m_old, bm)
        es = jnp.sum(jnp.exp(b - m_new), axis=0, keepdims=True)
        s_ref[...] = s_ref[...] * jnp.exp(m_old - m_new) + es
        m_ref[...] = m_new
        sx_ref[...] += jnp.sum(b, axis=0, keepdims=True)

    @pl.when(k == (_C // _BK) - 1)
    def _fin():
        lse = m_ref[...] + jnp.log(s_ref[...])
        row_loss = _EPS * (_C * lse - sx_ref[...]) + (_CONF - _EPS) * lse
        out_ref[0, 0] = jnp.sum(row_loss) * (1.0 / _N)


@functools.partial(
    pl.kernel,
    mesh=plsc.VectorSubcoreMesh(core_axis_name="c", subcore_axis_name="s"),
    out_type=jax.ShapeDtypeStruct((_NC, _L), jnp.float32),
    scratch_types=[
        pltpu.VMEM((_RW,), jnp.int32),
        pltpu.VMEM((_RW,), jnp.int32),
        pltpu.VMEM((_RW, 128), jnp.float32),
        pltpu.VMEM((_L,), jnp.float32),
        pltpu.VMEM((_NS, _L), jnp.float32),
        pltpu.VMEM_SHARED((_NS, _L), jnp.float32),
        pltpu.SemaphoreType.DMA,
    ],
)
def _sc_gather(phys, tgt, out, tgt_v, idx_v, spans_v, acc_v,
               stage_v, shared, sem):
    c = lax.axis_index("c")
    s = lax.axis_index("s")
    wid = s * _NC + c
    base = wid * _RW
    pltpu.sync_copy(tgt.at[pl.ds(base, _RW)], tgt_v)
    for g in range(_RW // _L):
        t16 = tgt_v[pl.ds(g * _L, _L)]
        rows = base + g * _L + lax.iota(jnp.int32, _L)
        span = (lax.shift_right_logical(t16, 3) * 64
                + lax.shift_right_logical(rows, 7) * 8
                + lax.bitwise_and(t16, 7))
        idx_v[pl.ds(g * _L, _L)] = span
    pltpu.async_copy(phys.at[idx_v], spans_v, sem).wait()
    lane0 = lax.bitwise_and(wid, 3) * 32
    lane16 = lax.iota(jnp.int32, _L)
    acc = jnp.zeros((_L,), jnp.float32)
    for j in range(_RW):
        chunk = spans_v[j, pl.ds(lane0 + (j // _L) * _L, _L)]
        acc = acc + jnp.where(lane16 == (j % _L), chunk, 0.0)
    acc_v[...] = acc
    pltpu.sync_copy(acc_v, shared.at[s])
    plsc.subcore_barrier()

    @pl.when(s == 0)
    def _reduce():
        pltpu.sync_copy(shared, stage_v)
        tot = jnp.zeros((_L,), jnp.float32)
        for j in range(_NS):
            tot = tot + stage_v[j]
        acc_v[...] = tot
        pltpu.sync_copy(acc_v, out.at[c])


def kernel(pred, target):
    tgt = target.astype(jnp.int32)
    xt = pred.T
    dense = pl.pallas_call(
        _tc_body,
        grid=(_C // _BK,),
        in_specs=[pl.BlockSpec((_BK, _N), lambda k: (k, 0))],
        out_specs=pl.BlockSpec(memory_space=pltpu.SMEM,
                               block_shape=(1, 1),
                               index_map=lambda k: (0, 0)),
        out_shape=jax.ShapeDtypeStruct((1, 1), jnp.float32),
        scratch_shapes=[
            pltpu.VMEM((1, _N), jnp.float32),
            pltpu.VMEM((1, _N), jnp.float32),
            pltpu.VMEM((1, _N), jnp.float32),
        ],
        compiler_params=pltpu.CompilerParams(
            dimension_semantics=("arbitrary",),
        ),
    )(xt)
    phys = jnp.transpose(pred.reshape(8, 128, _C // 8, 8),
                         (2, 0, 3, 1)).reshape(_C * 8, 128)
    partials = _sc_gather(phys, tgt)
    bsum = jnp.sum(partials)
    return dense[0, 0] - (_CONF - _EPS) * (bsum * (1.0 / _N))
